# baseline (speedup 1.0000x reference)
import jax
import jax.numpy as jnp
from jax import lax
from jax.experimental import pallas as pl
from jax.experimental.pallas import tpu as pltpu
from jax.experimental.pallas import tpu_sc as plsc

_K = 4
_G = 100000
_IN_DIM = 26
_OUT_DIM = 64
_NKNOTS = _G + 2 * _K - 1
_KNOTS_PAD = 100032
_L = 16
_NBUF = 3
_TAIL0 = _G - 32
_B1MAX = _TAIL0 - 256


def _sc_kan(x_hbm, w_hbm, silu_hbm, tail_hbm, knots_hbm, out_hbm,
            x_v, win_v, acc_v, col_bufs, tail_v, silu_v, sems, aux_sem):
    cid = lax.axis_index("c")
    sid = lax.axis_index("s")

    @pl.when(jnp.logical_and(cid == 0, sid == 0))
    def _work():
        pltpu.sync_copy(x_hbm, x_v)
        lanes = lax.iota(jnp.int32, _L)

        silu_cp = pltpu.async_copy(silu_hbm, silu_v, aux_sem)

        ic_groups, b8_groups = [], []
        win_copies = []
        for v in range(2):
            xv = x_v[pl.ds(v * _L, _L)]
            m_a = ((xv + 1.0) * (_G / 2.0)).astype(jnp.int32)
            ic = jnp.clip(m_a + (_K - 1), 13, _NKNOTS - _K - 1)
            b8 = jnp.bitwise_and(ic - 5, ~7)
            ic_groups.append(ic)
            b8_groups.append(b8)
            for cl in range(_L):
                chn = v * _L + cl
                if chn >= _IN_DIM:
                    break
                b8_s = lax.reduce_max(jnp.where(lanes == cl, b8, 0), axes=(0,))
                b8_s = pl.multiple_of(b8_s, 8)
                win_copies.append(pltpu.async_copy(
                    knots_hbm.at[pl.ds(b8_s, 24)],
                    win_v.at[pl.ds(chn * 32, 24)], sems[0]))
        for cp in win_copies:
            cp.wait()

        i_groups, taps_groups = [], []
        for v in range(2):
            xv = x_v[pl.ds(v * _L, _L)]
            ch = lanes + (v * _L)
            ic, b8 = ic_groups[v], b8_groups[v]
            wbase = ch * 32 - b8

            def tkn(e):
                return plsc.load_gather(
                    win_v, [jnp.clip(wbase + e, 0, _IN_DIM * 32 - 1)])

            i = (ic - 2) \
                + (tkn(ic - 1) <= xv).astype(jnp.int32) \
                + (tkn(ic) <= xv).astype(jnp.int32) \
                + (tkn(ic + 1) <= xv).astype(jnp.int32)
            i = jnp.clip(i, _K - 1, _NKNOTS - _K - 1)
            i_groups.append(i)

            w8 = [tkn(i - (_K - 1) + m) for m in range(2 * _K)]
            b = [jnp.ones((_L,), jnp.float32)]
            for d in range(1, _K):
                cols = []
                for j in range(d + 1):
                    m0 = (_K - 1) - d + j
                    den1 = w8[m0 + d] - w8[m0]
                    den2 = w8[m0 + d + 1] - w8[m0 + 1]
                    c1 = jnp.where(den1 > 0,
                                   (xv - w8[m0]) / jnp.where(den1 > 0, den1, 1.0),
                                   0.0)
                    c2 = jnp.where(den2 > 0,
                                   (w8[m0 + d + 1] - xv) / jnp.where(den2 > 0, den2, 1.0),
                                   0.0)
                    col = jnp.zeros((_L,), jnp.float32)
                    if j >= 1:
                        col = col + c1 * b[j - 1]
                    if j <= d - 1:
                        col = col + c2 * b[j]
                    cols.append(col)
                b = cols

            silu = xv / (1.0 + jnp.exp(-xv))
            taps_groups.append(b + [silu])

        def splat(chn, j):
            vec = taps_groups[chn // _L][j]
            s = lax.reduce_max(
                jnp.where(lanes == (chn % _L), vec, -jnp.inf), axes=(0,))
            return jnp.zeros((_L,), jnp.float32) + s

        def scal(chn, vec):
            return lax.reduce_max(
                jnp.where(lanes == (chn % _L), vec, 0), axes=(0,))

        acc = [jnp.zeros((_L,), jnp.float32) for _ in range(4)]
        i_s = [scal(chn, i_groups[chn // _L]) for chn in range(_IN_DIM)]
        b1_s = [pl.multiple_of(
                    jnp.minimum(jnp.bitwise_and(i_s[chn] - (_K - 1), ~127),
                                _B1MAX), 128)
                for chn in range(_IN_DIM)]

        def consume(pchn, bi):
            @pl.when(i_s[pchn] >= _TAIL0)
            def _():
                pltpu.sync_copy(tail_hbm.at[pchn], tail_v)
            for j in range(_K):
                c_j = i_s[pchn] - (_K - 1) + j
                off = jnp.clip(c_j - b1_s[pchn], 0, 255)
                offt = jnp.clip(c_j - _TAIL0, 0, 127)
                in_tail = c_j >= _TAIL0
                cvec = splat(pchn, j)
                for q in range(4):
                    idx_d = lanes + q * _L
                    g_ab = plsc.load_gather(
                        col_bufs[bi], [idx_d, jnp.zeros((_L,), jnp.int32) + off])
                    g_t = plsc.load_gather(
                        tail_v, [idx_d, jnp.zeros((_L,), jnp.int32) + offt])
                    val = jnp.where(in_tail, g_t, g_ab)
                    acc[q] = acc[q] + cvec * val

        pending = [None] * _NBUF
        for chn in range(_IN_DIM):
            bi = chn % _NBUF
            if pending[bi] is not None:
                (pchn, cpa, cpb) = pending[bi]
                cpa.wait()
                cpb.wait()
                consume(pchn, bi)
            cpa = pltpu.async_copy(
                w_hbm.at[chn, pl.ds(0, 64), pl.ds(b1_s[chn], 128)],
                col_bufs[bi].at[pl.ds(0, 64), pl.ds(0, 128)], sems[bi])
            cpb = pltpu.async_copy(
                w_hbm.at[chn, pl.ds(0, 64), pl.ds(b1_s[chn] + 128, 128)],
                col_bufs[bi].at[pl.ds(0, 64), pl.ds(128, 128)], sems[bi])
            pending[bi] = (chn, cpa, cpb)
        for bi in range(_NBUF):
            if pending[bi] is not None:
                (pchn, cpa, cpb) = pending[bi]
                cpa.wait()
                cpb.wait()
                consume(pchn, bi)

        silu_cp.wait()
        for chn in range(_IN_DIM):
            cvec = splat(chn, _K)
            for q in range(4):
                acc[q] = acc[q] + cvec * silu_v[chn, pl.ds(q * _L, _L)]

        for q in range(4):
            acc_v[pl.ds(q * _L, _L)] = acc[q]
        pltpu.sync_copy(acc_v, out_hbm)


@jax.jit
def kernel(x, w, knots):
    x_pad = jnp.zeros((2 * _L,), jnp.float32).at[:_IN_DIM].set(x)
    knots_pad = jnp.zeros((_KNOTS_PAD,), jnp.float32).at[:_NKNOTS].set(knots)
    wt = jnp.transpose(w, (1, 2, 0))
    w_silu = w[_G + _K - 1]
    w_tail = jnp.pad(jnp.transpose(w[_TAIL0:_G + _K], (1, 2, 0)),
                     ((0, 0), (0, 0), (0, 128 - (_G + _K - _TAIL0))))

    run = pl.kernel(
        _sc_kan,
        out_type=jax.ShapeDtypeStruct((_OUT_DIM,), jnp.float32),
        mesh=plsc.VectorSubcoreMesh(core_axis_name="c", subcore_axis_name="s"),
        scratch_types=[
            pltpu.VMEM((2 * _L,), jnp.float32),
            pltpu.VMEM((_IN_DIM * 32,), jnp.float32),
            pltpu.VMEM((_OUT_DIM,), jnp.float32),
            [pltpu.VMEM((64, 256), jnp.float32) for _ in range(_NBUF)],
            pltpu.VMEM((64, 128), jnp.float32),
            pltpu.VMEM((_IN_DIM, _OUT_DIM), jnp.float32),
            [pltpu.SemaphoreType.DMA for _ in range(_NBUF)],
            pltpu.SemaphoreType.DMA,
        ],
        compiler_params=pltpu.CompilerParams(needs_layout_passes=False),
    )
    return run(x_pad, wt, w_silu, w_tail, knots_pad)
